# Initial kernel scaffold; baseline (speedup 1.0000x reference)
import jax
import jax.numpy as jnp
from jax.experimental import pallas as pl


def kernel(input_ids, token_table, position_table, gamma, beta):
    raise NotImplementedError("write your pallas kernel here")


# speedup vs baseline: 1.9106x; 1.9106x over previous
import functools

import jax
import jax.numpy as jnp
from jax import lax
from jax.experimental import pallas as pl
from jax.experimental.pallas import tpu as pltpu
from jax.experimental.pallas import tpu_sc as plsc

_V, _H, _P, _B, _L = 100000, 128, 512, 1024, 200
_EPS = 1e-12

_NC, _NS, _LANES = 2, 16, 16
_NW = _NC * _NS
_T = _B * _L
_RPW = _T // _NW
_CH = 320
_NCHUNK = _RPW // _CH
_NB = _H // _LANES


def _rsqrt(x):
    xi = lax.bitcast_convert_type(x, jnp.int32)
    y = lax.bitcast_convert_type(jnp.int32(0x5F3759DF) - (xi >> 1), jnp.float32)
    for _ in range(3):
        y = y * (1.5 - 0.5 * x * y * y)
    return y


_mesh = plsc.VectorSubcoreMesh(core_axis_name="c", subcore_axis_name="s")


@functools.partial(
    pl.kernel,
    out_type=jax.ShapeDtypeStruct((_T, _H), jnp.float32),
    mesh=_mesh,
    scratch_types=[
        pltpu.VMEM((_L, _H), jnp.float32),
        pltpu.VMEM((_H,), jnp.float32),
        pltpu.VMEM((_H,), jnp.float32),
        pltpu.VMEM((_CH,), jnp.int32),
        pltpu.VMEM((_CH, _H), jnp.float32),
        pltpu.SemaphoreType.DMA,
    ],
    compiler_params=pltpu.CompilerParams(needs_layout_passes=False),
)
def _emb(ids_hbm, tok_hbm, pos_hbm, gamma_hbm, beta_hbm, out_hbm,
         pos_v, gamma_v, beta_v, idx_v, rows_v, sem):
    wid = lax.axis_index("s") * _NC + lax.axis_index("c")
    pltpu.sync_copy(pos_hbm.at[pl.ds(0, _L)], pos_v)
    pltpu.sync_copy(gamma_hbm, gamma_v)
    pltpu.sync_copy(beta_hbm, beta_v)
    w_base = wid * _RPW

    g_regs = [gamma_v[pl.ds(jb * _LANES, _LANES)] for jb in range(_NB)]
    b_regs = [beta_v[pl.ds(jb * _LANES, _LANES)] for jb in range(_NB)]

    def chunk_body(c, carry):
        base = w_base + c * _CH
        pltpu.sync_copy(ids_hbm.at[pl.ds(base, _CH)], idx_v)
        pltpu.async_copy(tok_hbm.at[idx_v], rows_v, sem).wait()

        def row_body(r, carry2):
            l = lax.rem(base + r, _L)
            xs = []
            acc = jnp.zeros((_LANES,), jnp.float32)
            acc2 = jnp.zeros((_LANES,), jnp.float32)
            for jb in range(_NB):
                x = (rows_v[r, pl.ds(jb * _LANES, _LANES)]
                     + pos_v[l, pl.ds(jb * _LANES, _LANES)])
                xs.append(x)
                acc = acc + x
                acc2 = acc2 + x * x
            s = jnp.sum(acc)
            ss = jnp.sum(acc2)
            mean = s * (1.0 / _H)
            var = ss * (1.0 / _H) - mean * mean
            rstd = _rsqrt(var + _EPS)
            shift = -mean * rstd
            for jb in range(_NB):
                xn = xs[jb] * rstd + shift
                rows_v[r, pl.ds(jb * _LANES, _LANES)] = xn * g_regs[jb] + b_regs[jb]
            return carry2

        lax.fori_loop(0, _CH, row_body, 0, unroll=2)
        pltpu.sync_copy(rows_v, out_hbm.at[pl.ds(base, _CH)])
        return carry

    lax.fori_loop(0, _NCHUNK, chunk_body, 0)


def kernel(input_ids, token_table, position_table, gamma, beta):
    ids_flat = input_ids.reshape(-1)
    out = _emb(ids_flat, token_table, position_table, gamma, beta)
    return out.reshape(_B, _L, _H)
